# baseline (speedup 1.0000x reference)
import functools

import jax
import jax.numpy as jnp
from jax import lax
from jax.experimental import pallas as pl
from jax.experimental.pallas import tpu as pltpu
from jax.experimental.pallas import tpu_sc as plsc

_N = 10000
_D = 128
_H = 128
_E = 320000
_EPS = 1e-5
_SLOPE = 0.01

_NC = 2
_NS = 16
_NW = _NC * _NS
_CHUNK = 128
_CPT = 80
_HCPT = 40
_EPT = _CHUNK * _CPT
_EP = _EPT * _NW
_NPAD = 10112
_RPS = _NPAD // _NS
_PAD_DST = _N

_mesh = plsc.VectorSubcoreMesh(core_axis_name="c", subcore_axis_name="s")


def _agg_body(h_hbm, src_hbm, dst_hbm, z_hbm,
              agg_out, src_v, dst_v, rows0, rows1, agg_sh, sem0, sem1):
    c = lax.axis_index("c")
    s = lax.axis_index("s")
    w = c * _NS + s
    pltpu.sync_copy(z_hbm.at[pl.ds(s * _RPS, _RPS)], agg_sh.at[pl.ds(s * _RPS, _RPS)])
    plsc.subcore_barrier()

    for half in range(_CPT // _HCPT):
        pltpu.sync_copy(src_hbm.at[w, pl.ds(half * _HCPT, _HCPT)], src_v)
        pltpu.sync_copy(dst_hbm.at[w, pl.ds(half * _HCPT, _HCPT)], dst_v)

        def step(j, carry):
            pltpu.async_copy(h_hbm.at[src_v.at[j]], rows0, sem0).wait()
            pltpu.sync_copy(rows0, agg_sh.at[dst_v.at[j]], add=True)
            return carry

        lax.fori_loop(0, _HCPT, step, 0)
    plsc.subcore_barrier()
    pltpu.sync_copy(agg_sh.at[pl.ds(s * _RPS, _RPS)], agg_out.at[c, pl.ds(s * _RPS, _RPS)])


def _cnt_body(dst_hbm, z_hbm, ones_hbm,
              cnt_out, dst_v, ones_v, cnt_sh, sem):
    c = lax.axis_index("c")
    s = lax.axis_index("s")
    w = c * _NS + s
    pltpu.sync_copy(z_hbm.at[pl.ds(s * _RPS, _RPS)], cnt_sh.at[pl.ds(s * _RPS, _RPS)])
    pltpu.sync_copy(ones_hbm, ones_v)
    pltpu.sync_copy(dst_hbm.at[w], dst_v)
    plsc.subcore_barrier()

    def step(j, carry):
        pltpu.sync_copy(ones_v, cnt_sh.at[dst_v.at[j]], add=True)
        return carry

    lax.fori_loop(0, _CPT, step, 0)
    plsc.subcore_barrier()
    pltpu.sync_copy(cnt_sh.at[pl.ds(s * _RPS, _RPS)],
                    cnt_out.at[c, pl.ds(s * _RPS, _RPS)])


_cnt = functools.partial(
    pl.kernel,
    mesh=_mesh,
    out_type=jax.ShapeDtypeStruct((_NC, _NPAD, _D), jnp.float32),
    scratch_types=[
        pltpu.VMEM((_CPT, _CHUNK), jnp.int32),
        pltpu.VMEM((_CHUNK, _D), jnp.float32),
        pltpu.VMEM_SHARED((_NPAD, _D), jnp.float32),
        pltpu.SemaphoreType.DMA,
    ],
)(_cnt_body)

_agg = functools.partial(
    pl.kernel,
    mesh=_mesh,
    out_type=jax.ShapeDtypeStruct((_NC, _NPAD, _D), jnp.float32),
    scratch_types=[
        pltpu.VMEM((_HCPT, _CHUNK), jnp.int32),
        pltpu.VMEM((_HCPT, _CHUNK), jnp.int32),
        pltpu.VMEM((_CHUNK, _D), jnp.float32),
        pltpu.VMEM((_CHUNK, _D), jnp.float32),
        pltpu.VMEM_SHARED((_NPAD, _D), jnp.float32),
        pltpu.SemaphoreType.DMA,
        pltpu.SemaphoreType.DMA,
    ],
)(_agg_body)


def _dense_body(with_bn, aggp_ref, cntp_ref, h_ref, wl_ref, wr_ref, b_ref,
                gamma_ref, beta_ref, out_ref):
    agg = aggp_ref[0, : _N, :] + aggp_ref[1, : _N, :]
    cnt = cntp_ref[0, : _N, 0:1] + cntp_ref[1, : _N, 0:1]
    mean = agg / jnp.clip(cnt, 1.0, None)
    lin = (jnp.dot(mean, wl_ref[...], preferred_element_type=jnp.float32)
           + b_ref[...]
           + jnp.dot(h_ref[...], wr_ref[...], preferred_element_type=jnp.float32))
    if with_bn:
        m = jnp.mean(lin, axis=0, keepdims=True)
        v = jnp.mean((lin - m) ** 2, axis=0, keepdims=True)
        lin = gamma_ref[...] * (lin - m) / jnp.sqrt(v + _EPS) + beta_ref[...]
        lin = jnp.where(lin >= 0, lin, _SLOPE * lin)
    out_ref[...] = lin


def _dense(with_bn, aggp, cntp, h, wl, wr, b, gamma, beta):
    return pl.pallas_call(
        functools.partial(_dense_body, with_bn),
        out_shape=jax.ShapeDtypeStruct((_N, _H), jnp.float32),
    )(aggp, cntp, h, wl, wr, b.reshape(1, _H), gamma.reshape(1, _H),
      beta.reshape(1, _H))


def kernel(x, edge_index, Wl0, Wr0, b0, Wl1, Wr1, b1, Wl2, Wr2, b2,
           gamma0, beta0, gamma1, beta1):
    pad = _EP - _E
    src = jnp.concatenate([edge_index[0], jnp.zeros((pad,), jnp.int32)])
    dst = jnp.concatenate([edge_index[1], jnp.full((pad,), _PAD_DST, jnp.int32)])
    src = src.reshape(_NW, _CPT, _CHUNK)
    dst = dst.reshape(_NW, _CPT, _CHUNK)
    z = jnp.zeros((_NPAD, _D), jnp.float32)
    ones128 = jnp.ones((_CHUNK, _D), jnp.float32)

    cntp = _cnt(dst, z, ones128)
    aggp = _agg(x, src, dst, z)
    h1 = _dense(True, aggp, cntp, x, Wl0, Wr0, b0, gamma0, beta0)
    aggp = _agg(h1, src, dst, z)
    h2 = _dense(True, aggp, cntp, h1, Wl1, Wr1, b1, gamma1, beta1)
    aggp = _agg(h2, src, dst, z)
    one = jnp.ones((_H,), jnp.float32)
    zero = jnp.zeros((_H,), jnp.float32)
    return _dense(False, aggp, cntp, h2, Wl2, Wr2, b2, one, zero)


# speedup vs baseline: 1.0498x; 1.0498x over previous
import functools

import jax
import jax.numpy as jnp
from jax import lax
from jax.experimental import pallas as pl
from jax.experimental.pallas import tpu as pltpu
from jax.experimental.pallas import tpu_sc as plsc

_N = 10000
_D = 128
_H = 128
_E = 320000
_EPS = 1e-5
_SLOPE = 0.01

_NC = 2
_NS = 16
_NW = _NC * _NS
_CHUNK = 128
_CPT = 80
_HCPT = 40
_EPT = _CHUNK * _CPT
_EP = _EPT * _NW
_NPAD = 10112
_RPS = _NPAD // _NS
_PAD_DST = _N

_mesh = plsc.VectorSubcoreMesh(core_axis_name="c", subcore_axis_name="s")


def _unpack_chunk(packed_v, j, src_idx, dst_idx):
    for k in range(_CHUNK // 16):
        p = packed_v[j, pl.ds(16 * k, 16)]
        src_idx[pl.ds(16 * k, 16)] = p & 0xFFFF
        dst_idx[pl.ds(16 * k, 16)] = p >> 16


def _agg_body(h_hbm, packed_hbm, z_hbm,
              agg_out, packed_v, si0, di0, si1, di1, rows0, rows1,
              agg_sh, sem0, sem1):
    c = lax.axis_index("c")
    s = lax.axis_index("s")
    w = c * _NS + s
    pltpu.sync_copy(z_hbm.at[pl.ds(s * _RPS, _RPS)], agg_sh.at[pl.ds(s * _RPS, _RPS)])
    pltpu.sync_copy(packed_hbm.at[w], packed_v)
    plsc.subcore_barrier()

    def step(g, carry):
        j0 = 2 * g
        j1 = j0 + 1
        _unpack_chunk(packed_v, j0, si0, di0)
        cp0 = pltpu.async_copy(h_hbm.at[si0], rows0, sem0)
        _unpack_chunk(packed_v, j1, si1, di1)
        cp1 = pltpu.async_copy(h_hbm.at[si1], rows1, sem1)
        cp0.wait()
        pltpu.sync_copy(rows0, agg_sh.at[di0], add=True)
        cp1.wait()
        pltpu.sync_copy(rows1, agg_sh.at[di1], add=True)
        return carry

    lax.fori_loop(0, _CPT // 2, step, 0)
    plsc.subcore_barrier()
    pltpu.sync_copy(agg_sh.at[pl.ds(s * _RPS, _RPS)], agg_out.at[c, pl.ds(s * _RPS, _RPS)])


def _cnt_body(dst_hbm, z_hbm, ones_hbm,
              cnt_out, dst_v, ones_v, cnt_sh, sem):
    c = lax.axis_index("c")
    s = lax.axis_index("s")
    w = c * _NS + s
    pltpu.sync_copy(z_hbm.at[pl.ds(s * _RPS, _RPS)], cnt_sh.at[pl.ds(s * _RPS, _RPS)])
    pltpu.sync_copy(ones_hbm, ones_v)
    pltpu.sync_copy(dst_hbm.at[w], dst_v)
    plsc.subcore_barrier()

    def step(j, carry):
        pltpu.sync_copy(ones_v, cnt_sh.at[dst_v.at[j]], add=True)
        return carry

    lax.fori_loop(0, _CPT, step, 0)
    plsc.subcore_barrier()
    pltpu.sync_copy(cnt_sh.at[pl.ds(s * _RPS, _RPS)],
                    cnt_out.at[c, pl.ds(s * _RPS, _RPS)])


_cnt = functools.partial(
    pl.kernel,
    mesh=_mesh,
    out_type=jax.ShapeDtypeStruct((_NC, _NPAD, _D), jnp.float32),
    scratch_types=[
        pltpu.VMEM((_CPT, _CHUNK), jnp.int32),
        pltpu.VMEM((_CHUNK, _D), jnp.float32),
        pltpu.VMEM_SHARED((_NPAD, _D), jnp.float32),
        pltpu.SemaphoreType.DMA,
    ],
)(_cnt_body)

_agg = functools.partial(
    pl.kernel,
    mesh=_mesh,
    out_type=jax.ShapeDtypeStruct((_NC, _NPAD, _D), jnp.float32),
    scratch_types=[
        pltpu.VMEM((_CPT, _CHUNK), jnp.int32),
        pltpu.VMEM((_CHUNK,), jnp.int32),
        pltpu.VMEM((_CHUNK,), jnp.int32),
        pltpu.VMEM((_CHUNK,), jnp.int32),
        pltpu.VMEM((_CHUNK,), jnp.int32),
        pltpu.VMEM((_CHUNK, _D), jnp.float32),
        pltpu.VMEM((_CHUNK, _D), jnp.float32),
        pltpu.VMEM_SHARED((_NPAD, _D), jnp.float32),
        pltpu.SemaphoreType.DMA,
        pltpu.SemaphoreType.DMA,
    ],
)(_agg_body)


def _dense_body(with_bn, aggp_ref, cntp_ref, h_ref, wl_ref, wr_ref, b_ref,
                gamma_ref, beta_ref, out_ref):
    agg = aggp_ref[0, : _N, :] + aggp_ref[1, : _N, :]
    cnt = cntp_ref[0, : _N, 0:1] + cntp_ref[1, : _N, 0:1]
    mean = agg / jnp.clip(cnt, 1.0, None)
    lin = (jnp.dot(mean, wl_ref[...], preferred_element_type=jnp.float32)
           + b_ref[...]
           + jnp.dot(h_ref[...], wr_ref[...], preferred_element_type=jnp.float32))
    if with_bn:
        m = jnp.mean(lin, axis=0, keepdims=True)
        v = jnp.mean((lin - m) ** 2, axis=0, keepdims=True)
        lin = gamma_ref[...] * (lin - m) / jnp.sqrt(v + _EPS) + beta_ref[...]
        lin = jnp.where(lin >= 0, lin, _SLOPE * lin)
    out_ref[...] = lin


def _dense(with_bn, aggp, cntp, h, wl, wr, b, gamma, beta):
    return pl.pallas_call(
        functools.partial(_dense_body, with_bn),
        out_shape=jax.ShapeDtypeStruct((_N, _H), jnp.float32),
    )(aggp, cntp, h, wl, wr, b.reshape(1, _H), gamma.reshape(1, _H),
      beta.reshape(1, _H))


def kernel(x, edge_index, Wl0, Wr0, b0, Wl1, Wr1, b1, Wl2, Wr2, b2,
           gamma0, beta0, gamma1, beta1):
    pad = _EP - _E
    src = jnp.concatenate([edge_index[0], jnp.zeros((pad,), jnp.int32)])
    dst = jnp.concatenate([edge_index[1], jnp.full((pad,), _PAD_DST, jnp.int32)])
    packed = (src | (dst << 16)).reshape(_NW, _CPT, _CHUNK)
    dst = dst.reshape(_NW, _CPT, _CHUNK)
    z = jnp.zeros((_NPAD, _D), jnp.float32)
    ones128 = jnp.ones((_CHUNK, _D), jnp.float32)

    cntp = _cnt(dst, z, ones128)
    aggp = _agg(x, packed, z)
    h1 = _dense(True, aggp, cntp, x, Wl0, Wr0, b0, gamma0, beta0)
    aggp = _agg(h1, packed, z)
    h2 = _dense(True, aggp, cntp, h1, Wl1, Wr1, b1, gamma1, beta1)
    aggp = _agg(h2, packed, z)
    one = jnp.ones((_H,), jnp.float32)
    zero = jnp.zeros((_H,), jnp.float32)
    return _dense(False, aggp, cntp, h2, Wl2, Wr2, b2, one, zero)


# speedup vs baseline: 1.5284x; 1.4559x over previous
import functools

import jax
import jax.numpy as jnp
from jax import lax
from jax.experimental import pallas as pl
from jax.experimental.pallas import tpu as pltpu
from jax.experimental.pallas import tpu_sc as plsc

_N = 10000
_D = 128
_H = 128
_E = 320000
_EPS = 1e-5
_SLOPE = 0.01

_NC = 2
_NS = 16
_NW = _NC * _NS
_CHUNK = 128
_CPT = 79
_EPT = _CHUNK * _CPT
_EP = _EPT * _NW
_NPAD = 10112
_RPS = _NPAD // _NS
_PAD_DST = _N

_mesh = plsc.VectorSubcoreMesh(core_axis_name="c", subcore_axis_name="s")


def _agg_body(h_hbm, src_hbm, dst_hbm, z_hbm,
              agg_out, src_v, dst_v, rows_v, agg_sh, sem):
    c = lax.axis_index("c")
    s = lax.axis_index("s")
    w = c * _NS + s
    pltpu.sync_copy(z_hbm.at[pl.ds(s * _RPS, _RPS)], agg_sh.at[pl.ds(s * _RPS, _RPS)])
    pltpu.sync_copy(src_hbm.at[w], src_v)
    pltpu.sync_copy(dst_hbm.at[w], dst_v)
    plsc.subcore_barrier()

    def step(j, carry):
        pltpu.async_copy(h_hbm.at[src_v.at[j]], rows_v, sem).wait()
        pltpu.sync_copy(rows_v, agg_sh.at[dst_v.at[j]], add=True)
        return carry

    lax.fori_loop(0, _CPT, step, 0)
    plsc.subcore_barrier()
    pltpu.sync_copy(agg_sh.at[pl.ds(s * _RPS, _RPS)], agg_out.at[c, pl.ds(s * _RPS, _RPS)])


def _cnt_body(dst_hbm, z_hbm, ones_hbm,
              cnt_out, dst_v, ones_v, cnt_sh, sem):
    c = lax.axis_index("c")
    s = lax.axis_index("s")
    w = c * _NS + s
    pltpu.sync_copy(z_hbm.at[pl.ds(s * _RPS, _RPS)], cnt_sh.at[pl.ds(s * _RPS, _RPS)])
    pltpu.sync_copy(ones_hbm, ones_v)
    pltpu.sync_copy(dst_hbm.at[w], dst_v)
    plsc.subcore_barrier()

    def step(j, carry):
        pltpu.sync_copy(ones_v, cnt_sh.at[dst_v.at[j]], add=True)
        return carry

    lax.fori_loop(0, _CPT, step, 0)
    plsc.subcore_barrier()
    pltpu.sync_copy(cnt_sh.at[pl.ds(s * _RPS, _RPS)],
                    cnt_out.at[c, pl.ds(s * _RPS, _RPS)])


_cnt = functools.partial(
    pl.kernel,
    mesh=_mesh,
    out_type=jax.ShapeDtypeStruct((_NC, _NPAD, _D), jnp.float32),
    scratch_types=[
        pltpu.VMEM((_CPT, _CHUNK), jnp.int32),
        pltpu.VMEM((_CHUNK, _D), jnp.float32),
        pltpu.VMEM_SHARED((_NPAD, _D), jnp.float32),
        pltpu.SemaphoreType.DMA,
    ],
)(_cnt_body)

_agg = functools.partial(
    pl.kernel,
    mesh=_mesh,
    out_type=jax.ShapeDtypeStruct((_NC, _NPAD, _D), jnp.float32),
    scratch_types=[
        pltpu.VMEM((_CPT, _CHUNK), jnp.int32),
        pltpu.VMEM((_CPT, _CHUNK), jnp.int32),
        pltpu.VMEM((_CHUNK, _D), jnp.float32),
        pltpu.VMEM_SHARED((_NPAD, _D), jnp.float32),
        pltpu.SemaphoreType.DMA,
    ],
)(_agg_body)


def _dense_body(with_bn, aggp_ref, cntp_ref, h_ref, wl_ref, wr_ref, b_ref,
                gamma_ref, beta_ref, out_ref):
    agg = aggp_ref[0, : _N, :] + aggp_ref[1, : _N, :]
    cnt = cntp_ref[0, : _N, 0:1] + cntp_ref[1, : _N, 0:1]
    mean = agg / jnp.clip(cnt, 1.0, None)
    lin = (jnp.dot(mean, wl_ref[...], preferred_element_type=jnp.float32)
           + b_ref[...]
           + jnp.dot(h_ref[...], wr_ref[...], preferred_element_type=jnp.float32))
    if with_bn:
        m = jnp.mean(lin, axis=0, keepdims=True)
        v = jnp.mean((lin - m) ** 2, axis=0, keepdims=True)
        lin = gamma_ref[...] * (lin - m) / jnp.sqrt(v + _EPS) + beta_ref[...]
        lin = jnp.where(lin >= 0, lin, _SLOPE * lin)
    out_ref[...] = lin


def _dense(with_bn, aggp, cntp, h, wl, wr, b, gamma, beta):
    return pl.pallas_call(
        functools.partial(_dense_body, with_bn),
        out_shape=jax.ShapeDtypeStruct((_N, _H), jnp.float32),
    )(aggp, cntp, h, wl, wr, b.reshape(1, _H), gamma.reshape(1, _H),
      beta.reshape(1, _H))


def kernel(x, edge_index, Wl0, Wr0, b0, Wl1, Wr1, b1, Wl2, Wr2, b2,
           gamma0, beta0, gamma1, beta1):
    pad = _EP - _E
    src = jnp.concatenate([edge_index[0], jnp.zeros((pad,), jnp.int32)])
    dst = jnp.concatenate([edge_index[1], jnp.full((pad,), _PAD_DST, jnp.int32)])
    src = src.reshape(_NW, _CPT, _CHUNK)
    dst = dst.reshape(_NW, _CPT, _CHUNK)
    z = jnp.zeros((_NPAD, _D), jnp.float32)
    ones128 = jnp.ones((_CHUNK, _D), jnp.float32)

    cntp = _cnt(dst, z, ones128)
    aggp = _agg(x, src, dst, z)
    h1 = _dense(True, aggp, cntp, x, Wl0, Wr0, b0, gamma0, beta0)
    aggp = _agg(h1, src, dst, z)
    h2 = _dense(True, aggp, cntp, h1, Wl1, Wr1, b1, gamma1, beta1)
    aggp = _agg(h2, src, dst, z)
    one = jnp.ones((_H,), jnp.float32)
    zero = jnp.zeros((_H,), jnp.float32)
    return _dense(False, aggp, cntp, h2, Wl2, Wr2, b2, one, zero)
